# Initial kernel scaffold; baseline (speedup 1.0000x reference)
import jax
import jax.numpy as jnp
from jax.experimental import pallas as pl


def kernel(tokens, emb, in_proj_w, conv_w, conv_b, x_proj_w, dt_proj_w, dt_proj_b, A_log, D, out_proj_w, fc_w, fc_b):
    raise NotImplementedError("write your pallas kernel here")


# speedup vs baseline: 24.5345x; 24.5345x over previous
import functools

import jax
import jax.numpy as jnp
from jax.experimental import pallas as pl
from jax.experimental.pallas import tpu as pltpu

B = 4
L = 2048
NUM_TOKENS = 1024
D_MODEL = 1024
D_STATE = 16
D_CONV = 4
D_INNER = 2048
DT_RANK = 64

T = 256
C = L // T


def _eprime_kernel(emb_ref, w_ref, out_ref):
    out_ref[...] = jnp.dot(emb_ref[...], w_ref[...],
                           preferred_element_type=jnp.float32)


def _main_kernel(tok_ref, eg_ref, cw_ref, cb_ref, xp_ref, dtw_ref, dtb_ref,
                 alog_ref, h_out_ref, xcl_out_ref,
                 xi_s, xc_s, delta_s, u_s, r_s, h_s):
    b = pl.program_id(0)
    c = pl.program_id(1)

    @pl.when(c == 0)
    def _():
        xi_s[5:8] = jnp.zeros((3, 1, D_INNER), jnp.float32)
        h_s[...] = jnp.zeros((D_STATE, D_INNER), jnp.float32)

    def gather_body(jj, _):
        for u in range(8):
            i = jj * 8 + u
            tok = tok_ref[b, c * T + i]
            xi_s[8 + i] = eg_ref[tok]
        return 0

    jax.lax.fori_loop(0, T // 8, gather_body, 0)

    xc3 = cb_ref[...]
    for k in range(D_CONV):
        xc3 = xc3 + xi_s[5 + k: 5 + k + T] * cw_ref[k:k + 1]
    xc3 = jax.nn.silu(xc3)
    xc_s[...] = xc3.reshape(T, D_INNER)

    tail = xi_s[T + 5: T + 8]
    xi_s[5:8] = tail

    xc = xc_s[...]
    dbc = jnp.dot(xc, xp_ref[...], preferred_element_type=jnp.float32)
    dt = dbc[:, :DT_RANK]

    pre = jnp.dot(dt, dtw_ref[...],
                  preferred_element_type=jnp.float32) + dtb_ref[...]
    delta = jax.nn.softplus(pre)
    delta_s[...] = delta
    u_s[...] = delta * xc

    row = jax.lax.broadcasted_iota(jnp.int32, (T, T), 0)
    col = jax.lax.broadcasted_iota(jnp.int32, (T, T), 1)
    tmat = (col > row).astype(jnp.float32)
    r_s[...] = jnp.dot(tmat, delta, preferred_element_type=jnp.float32)

    total = r_s[0:1, :] + delta_s[0:1, :]
    a_neg = -jnp.exp(alog_ref[...])
    decay = jnp.exp(a_neg * total)

    rr = r_s[...]
    uu = u_s[...]
    rows = []
    for n in range(D_STATE):
        w = jnp.exp(rr * a_neg[n:n + 1, :]) * uu * dbc[:, DT_RANK + n:DT_RANK + n + 1]
        rows.append(jnp.sum(w, axis=0, keepdims=True))
    hc = jnp.concatenate(rows, axis=0)

    h_s[...] = decay * h_s[...] + hc
    h_out_ref[0] = h_s[...]
    xcl_out_ref[...] = xc_s[pl.ds(T - 1, 1), :].reshape(1, 1, D_INNER)


def _epilogue_kernel(tokl_ref, emb_ref, wz_ref, h_ref, xcl_ref, cw_ref,
                     d_ref, wo_ref, fcw_ref, fcb_ref, out_ref):
    rows = [emb_ref[tokl_ref[0, bb]] for bb in range(B)]
    x_last = jnp.concatenate(rows, axis=0)
    z = jnp.dot(x_last, wz_ref[...], preferred_element_type=jnp.float32)
    xcl = xcl_ref[...]
    cm = jnp.dot(xcl, cw_ref[...], preferred_element_type=jnp.float32)
    hv = h_ref[...]
    y = jnp.sum(hv * cm[:, :, None], axis=1)
    y = y + d_ref[...] * xcl
    y = y * jax.nn.silu(z)
    o = jnp.dot(y, wo_ref[...], preferred_element_type=jnp.float32)
    out_ref[...] = jnp.dot(o, fcw_ref[...],
                           preferred_element_type=jnp.float32) + fcb_ref[...]


@jax.jit
def kernel(tokens, emb, in_proj_w, conv_w, conv_b, x_proj_w, dt_proj_w,
           dt_proj_b, A_log, D, out_proj_w, fc_w, fc_b):
    tokens = tokens.astype(jnp.int32)
    w_xi = in_proj_w[:, :D_INNER]
    w_z = in_proj_w[:, D_INNER:]

    eprime = pl.pallas_call(
        _eprime_kernel,
        grid=(2,),
        in_specs=[
            pl.BlockSpec((NUM_TOKENS // 2, D_MODEL), lambda i: (i, 0)),
            pl.BlockSpec((D_MODEL, D_INNER), lambda i: (0, 0)),
        ],
        out_specs=pl.BlockSpec((NUM_TOKENS // 2, D_INNER), lambda i: (i, 0)),
        out_shape=jax.ShapeDtypeStruct((NUM_TOKENS, D_INNER), jnp.float32),
        compiler_params=pltpu.CompilerParams(
            dimension_semantics=("parallel",)),
    )(emb, w_xi)

    eg = eprime.reshape(NUM_TOKENS, 1, D_INNER)
    cw3 = conv_w.T.reshape(D_CONV, 1, D_INNER)
    cb3 = conv_b.reshape(1, 1, D_INNER)
    dtb2 = dt_proj_b.reshape(1, D_INNER)
    alogt = A_log.T

    h_final, xcl = pl.pallas_call(
        _main_kernel,
        grid=(B, C),
        in_specs=[
            pl.BlockSpec(memory_space=pltpu.SMEM),
            pl.BlockSpec((NUM_TOKENS, 1, D_INNER), lambda b, c: (0, 0, 0)),
            pl.BlockSpec((D_CONV, 1, D_INNER), lambda b, c: (0, 0, 0)),
            pl.BlockSpec((1, 1, D_INNER), lambda b, c: (0, 0, 0)),
            pl.BlockSpec((D_INNER, DT_RANK + 2 * D_STATE),
                         lambda b, c: (0, 0)),
            pl.BlockSpec((DT_RANK, D_INNER), lambda b, c: (0, 0)),
            pl.BlockSpec((1, D_INNER), lambda b, c: (0, 0)),
            pl.BlockSpec((D_STATE, D_INNER), lambda b, c: (0, 0)),
        ],
        out_specs=[
            pl.BlockSpec((1, D_STATE, D_INNER), lambda b, c: (b, 0, 0)),
            pl.BlockSpec((1, 1, D_INNER), lambda b, c: (b, 0, 0)),
        ],
        out_shape=[
            jax.ShapeDtypeStruct((B, D_STATE, D_INNER), jnp.float32),
            jax.ShapeDtypeStruct((B, 1, D_INNER), jnp.float32),
        ],
        scratch_shapes=[
            pltpu.VMEM((T + 8, 1, D_INNER), jnp.float32),
            pltpu.VMEM((T, D_INNER), jnp.float32),
            pltpu.VMEM((T, D_INNER), jnp.float32),
            pltpu.VMEM((T, D_INNER), jnp.float32),
            pltpu.VMEM((T, D_INNER), jnp.float32),
            pltpu.VMEM((D_STATE, D_INNER), jnp.float32),
        ],
        compiler_params=pltpu.CompilerParams(
            dimension_semantics=("parallel", "arbitrary")),
    )(tokens, eg, cw3, cb3, x_proj_w, dt_proj_w, dtb2, alogt)

    tokl = tokens[:, -1].reshape(1, B)
    emb3 = emb.reshape(NUM_TOKENS, 1, D_MODEL)
    xcl2 = xcl.reshape(B, D_INNER)
    cw_c = x_proj_w[:, DT_RANK + D_STATE:]
    d2 = D.reshape(1, D_INNER)
    fcb2 = fc_b.reshape(1, NUM_TOKENS)

    logits = pl.pallas_call(
        _epilogue_kernel,
        in_specs=[
            pl.BlockSpec(memory_space=pltpu.SMEM),
            pl.BlockSpec(memory_space=pltpu.VMEM),
            pl.BlockSpec(memory_space=pltpu.VMEM),
            pl.BlockSpec(memory_space=pltpu.VMEM),
            pl.BlockSpec(memory_space=pltpu.VMEM),
            pl.BlockSpec(memory_space=pltpu.VMEM),
            pl.BlockSpec(memory_space=pltpu.VMEM),
            pl.BlockSpec(memory_space=pltpu.VMEM),
            pl.BlockSpec(memory_space=pltpu.VMEM),
            pl.BlockSpec(memory_space=pltpu.VMEM),
        ],
        out_specs=pl.BlockSpec(memory_space=pltpu.VMEM),
        out_shape=jax.ShapeDtypeStruct((B, NUM_TOKENS), jnp.float32),
    )(tokl, emb3, w_z, h_final, xcl2, cw_c, d2, out_proj_w, fc_w, fcb2)

    return logits
